# Initial kernel scaffold; baseline (speedup 1.0000x reference)
import jax
import jax.numpy as jnp
from jax.experimental import pallas as pl


def kernel(inputs, targets):
    raise NotImplementedError("write your pallas kernel here")


# speedup vs baseline: 3.2340x; 3.2340x over previous
import jax
import jax.numpy as jnp
from jax.experimental import pallas as pl
from jax.experimental.pallas import tpu as pltpu

_MARGIN = 0.3


def _prep_kernel(x_ref, x16_ref, sqh_ref):
    x = x_ref[...]
    x16_ref[...] = x.astype(jnp.bfloat16)
    sqh_ref[...] = 0.5 * jnp.sum(x * x, axis=1, keepdims=True)


def _mine_kernel(a_ref, b_ref, sqh_ref, ta_ref, tb_ref, j_ref, k_ref,
                 mx, mn, cj, ck, tp, tn):
    c = pl.program_id(1)
    nc = pl.num_programs(1)

    @pl.when(c == 0)
    def _init():
        mx[...] = jnp.full(mx.shape, -jnp.inf, jnp.float32)
        mn[...] = jnp.full(mn.shape, jnp.inf, jnp.float32)
        cj[...] = jnp.zeros(cj.shape, jnp.float32)
        ck[...] = jnp.zeros(ck.shape, jnp.float32)
        tp[...] = jnp.zeros(tp.shape, jnp.float32)
        tn[...] = jnp.zeros(tn.shape, jnp.float32)

    a = a_ref[...]
    ta = ta_ref[0]

    one = jnp.float32(1.0)
    zero = jnp.float32(0.0)
    big = jnp.float32(3.0e38)

    mx_v = mx[...]
    mn_v = mn[...]
    cj_v = cj[...]
    ck_v = ck[...]
    tp_v = tp[...]
    tn_v = tn[...]

    bc_full = b_ref.shape[0]
    chunk = 256
    for u in range(bc_full // chunk):
        b = b_ref[u * chunk:(u + 1) * chunk, :]
        tb = tb_ref[0, u * chunk:(u + 1) * chunk, :]
        sqbh = sqh_ref[0, u * chunk:(u + 1) * chunk, :]

        g = jax.lax.dot_general(b, a, (((1,), (1,)), ((), ())),
                                preferred_element_type=jnp.float32)
        s = sqbh - g

        m = tb == ta
        mf = jnp.where(m, one, zero)
        pos = jnp.where(m, s, -jnp.inf)
        neg = jnp.where(m, jnp.inf, s)
        bmax = jnp.max(pos, axis=0, keepdims=True)
        bmin = jnp.min(neg, axis=0, keepdims=True)

        rowf = jax.lax.broadcasted_iota(jnp.int32, m.shape, 0).astype(jnp.float32)
        prow = jnp.min(jnp.where(pos == bmax, rowf, big), axis=0, keepdims=True)
        nrow = jnp.min(jnp.where(neg == bmin, rowf, big), axis=0, keepdims=True)

        lep = jnp.where(rowf <= prow, one, zero)
        len_ = jnp.where(rowf <= nrow, one, zero)
        cpos = jnp.sum(lep * mf, axis=0, keepdims=True)
        cneg_m = jnp.sum(len_ * mf, axis=0, keepdims=True)
        cneg = (nrow + one) - cneg_m
        bp = jnp.sum(mf, axis=0, keepdims=True)

        updp = bmax > mx_v
        cj_v = jnp.where(updp, tp_v + cpos, cj_v)
        mx_v = jnp.where(updp, bmax, mx_v)
        updn = bmin < mn_v
        ck_v = jnp.where(updn, tn_v + cneg, ck_v)
        mn_v = jnp.where(updn, bmin, mn_v)
        tp_v = tp_v + bp
        tn_v = tn_v + (jnp.float32(chunk) - bp)

    mx[...] = mx_v
    mn[...] = mn_v
    cj[...] = cj_v
    ck[...] = ck_v
    tp[...] = tp_v
    tn[...] = tn_v

    @pl.when(c == nc - 1)
    def _fin():
        j_ref[0] = cj[...].astype(jnp.int32) - 1
        k_ref[0] = ck[...].astype(jnp.int32) - 1


def _loss_kernel(xi_ref, xj_ref, xk_ref, sqi_ref, sqj_ref, sqk_ref,
                 rank_ref, refm_ref, accr, accf):
    st = pl.program_id(0)
    ns = pl.num_programs(0)

    @pl.when(st == 0)
    def _init():
        accr[...] = jnp.zeros(accr.shape, jnp.float32)
        accf[...] = jnp.zeros(accf.shape, jnp.float32)

    xi = xi_ref[...].astype(jnp.float32)
    xj = xj_ref[...].astype(jnp.float32)
    xk = xk_ref[...].astype(jnp.float32)
    sqi = sqi_ref[...]
    sqj = sqj_ref[...]
    sqk = sqk_ref[...]
    pij = jnp.sum(xi * xj, axis=1, keepdims=True)
    pik = jnp.sum(xi * xk, axis=1, keepdims=True)
    pjk = jnp.sum(xj * xk, axis=1, keepdims=True)
    eps = jnp.float32(1e-12)
    dap = jnp.sqrt(jnp.maximum(2.0 * (sqi + sqj - pij), eps))
    dan = jnp.sqrt(jnp.maximum(2.0 * (sqi + sqk - pik), eps))
    djk = jnp.sqrt(jnp.maximum(2.0 * (sqj + sqk - pjk), eps))
    rank = jnp.maximum(dap - dan + jnp.float32(_MARGIN), 0.0)
    refv = jnp.abs(dan - djk)
    accr[...] = accr[...] + jnp.sum(rank, axis=0, keepdims=True)[:, :1]
    accf[...] = accf[...] + jnp.sum(refv, axis=0, keepdims=True)[:, :1]

    @pl.when(st == ns - 1)
    def _fin():
        rank_ref[...] = accr[...]
        refm_ref[...] = accf[...]


def kernel(inputs, targets):
    x = inputs.astype(jnp.float32)
    t = targets.astype(jnp.int32)
    n, d = x.shape

    bp_ = min(1024, n)
    x16, sqh = pl.pallas_call(
        _prep_kernel,
        grid=(n // bp_,),
        in_specs=[pl.BlockSpec((bp_, d), lambda i: (i, 0))],
        out_specs=[
            pl.BlockSpec((bp_, d), lambda i: (i, 0)),
            pl.BlockSpec((bp_, 1), lambda i: (i, 0)),
        ],
        out_shape=[
            jax.ShapeDtypeStruct((n, d), jnp.bfloat16),
            jax.ShapeDtypeStruct((n, 1), jnp.float32),
        ],
        compiler_params=pltpu.CompilerParams(
            dimension_semantics=("parallel",),
        ),
        name="prep_norms",
    )(x)

    br = min(2048, n)
    bcnd = min(1024, n)
    nrb = n // br
    ncb = n // bcnd
    ta3 = t.reshape(nrb, 1, br)
    tb3 = t.reshape(ncb, bcnd, 1)
    sqh3 = sqh.reshape(ncb, bcnd, 1)

    j3, k3 = pl.pallas_call(
        _mine_kernel,
        grid=(nrb, ncb),
        in_specs=[
            pl.BlockSpec((br, d), lambda i, c: (i, 0)),
            pl.BlockSpec((bcnd, d), lambda i, c: (c, 0)),
            pl.BlockSpec((1, bcnd, 1), lambda i, c: (c, 0, 0)),
            pl.BlockSpec((1, 1, br), lambda i, c: (i, 0, 0)),
            pl.BlockSpec((1, bcnd, 1), lambda i, c: (c, 0, 0)),
        ],
        out_specs=[
            pl.BlockSpec((1, 1, br), lambda i, c: (i, 0, 0)),
            pl.BlockSpec((1, 1, br), lambda i, c: (i, 0, 0)),
        ],
        out_shape=[
            jax.ShapeDtypeStruct((nrb, 1, br), jnp.int32),
            jax.ShapeDtypeStruct((nrb, 1, br), jnp.int32),
        ],
        scratch_shapes=[
            pltpu.VMEM((1, br), jnp.float32),
            pltpu.VMEM((1, br), jnp.float32),
            pltpu.VMEM((1, br), jnp.float32),
            pltpu.VMEM((1, br), jnp.float32),
            pltpu.VMEM((1, br), jnp.float32),
            pltpu.VMEM((1, br), jnp.float32),
        ],
        compiler_params=pltpu.CompilerParams(
            dimension_semantics=("parallel", "arbitrary"),
        ),
        name="hard_mine",
    )(x16, x16, sqh3, ta3, tb3)

    j = j3.reshape(n)
    k = k3.reshape(n)
    xj16 = jnp.take(x16, j, axis=0, mode="wrap")
    xk16 = jnp.take(x16, k, axis=0, mode="wrap")
    sqf = sqh.reshape(n)
    sqj_g = jnp.take(sqf, j, mode="wrap").reshape(n, 1)
    sqk_g = jnp.take(sqf, k, mode="wrap").reshape(n, 1)

    b2 = min(1024, n)
    ns2 = n // b2
    rank_s, ref_s = pl.pallas_call(
        _loss_kernel,
        grid=(ns2,),
        in_specs=[
            pl.BlockSpec((b2, d), lambda s: (s, 0)),
            pl.BlockSpec((b2, d), lambda s: (s, 0)),
            pl.BlockSpec((b2, d), lambda s: (s, 0)),
            pl.BlockSpec((b2, 1), lambda s: (s, 0)),
            pl.BlockSpec((b2, 1), lambda s: (s, 0)),
            pl.BlockSpec((b2, 1), lambda s: (s, 0)),
        ],
        out_specs=[
            pl.BlockSpec((1, 1), lambda s: (0, 0)),
            pl.BlockSpec((1, 1), lambda s: (0, 0)),
        ],
        out_shape=[
            jax.ShapeDtypeStruct((1, 1), jnp.float32),
            jax.ShapeDtypeStruct((1, 1), jnp.float32),
        ],
        scratch_shapes=[
            pltpu.VMEM((1, 1), jnp.float32),
            pltpu.VMEM((1, 1), jnp.float32),
        ],
        compiler_params=pltpu.CompilerParams(
            dimension_semantics=("arbitrary",),
        ),
        name="mined_loss",
    )(x16, xj16, xk16, sqh, sqj_g, sqk_g)

    return (rank_s[0, 0] + ref_s[0, 0]) / jnp.float32(n)


# speedup vs baseline: 3.4237x; 1.0586x over previous
import jax
import jax.numpy as jnp
from jax.experimental import pallas as pl
from jax.experimental.pallas import tpu as pltpu

_MARGIN = 0.3


def _prep_kernel(x_ref, x16_ref, sqh_ref):
    x = x_ref[...]
    x16_ref[...] = x.astype(jnp.bfloat16)
    sqh_ref[...] = 0.5 * jnp.sum(x * x, axis=1, keepdims=True)


def _mine_kernel(a_ref, b_ref, sqh_ref, ta_ref, tb_ref, j_ref, k_ref,
                 mx, mn, cj, ck, tp, tn):
    c = pl.program_id(1)
    nc = pl.num_programs(1)

    @pl.when(c == 0)
    def _init():
        mx[...] = jnp.full(mx.shape, -jnp.inf, jnp.float32)
        mn[...] = jnp.full(mn.shape, jnp.inf, jnp.float32)
        cj[...] = jnp.zeros(cj.shape, jnp.float32)
        ck[...] = jnp.zeros(ck.shape, jnp.float32)
        tp[...] = jnp.zeros(tp.shape, jnp.float32)
        tn[...] = jnp.zeros(tn.shape, jnp.float32)

    a = a_ref[...]
    ta = ta_ref[0]

    one = jnp.float32(1.0)
    zero = jnp.float32(0.0)
    big = jnp.float32(3.0e38)

    mx_v = mx[...]
    mn_v = mn[...]
    cj_v = cj[...]
    ck_v = ck[...]
    tp_v = tp[...]
    tn_v = tn[...]

    bc_full = b_ref.shape[0]
    chunk = 256
    for u in range(bc_full // chunk):
        b = b_ref[u * chunk:(u + 1) * chunk, :]
        tb = tb_ref[0, u * chunk:(u + 1) * chunk, :]
        sqbh = sqh_ref[0, u * chunk:(u + 1) * chunk, :]

        g = jax.lax.dot_general(b, a, (((1,), (1,)), ((), ())),
                                preferred_element_type=jnp.float32)
        s = sqbh - g

        m = tb == ta
        pos = jnp.where(m, s, -jnp.inf)
        neg = jnp.where(m, jnp.inf, s)
        bmax = jnp.max(pos, axis=0, keepdims=True)
        bmin = jnp.min(neg, axis=0, keepdims=True)

        rowf = jax.lax.broadcasted_iota(jnp.int32, m.shape, 0).astype(jnp.float32)
        prow = jnp.min(jnp.where(pos == bmax, rowf, big), axis=0, keepdims=True)
        nrow = jnp.min(jnp.where(neg == bmin, rowf, big), axis=0, keepdims=True)

        lep = jnp.where(rowf <= prow, one, zero)
        len_ = jnp.where(rowf <= nrow, one, zero)
        cpos = jnp.sum(jnp.where(m, lep, zero), axis=0, keepdims=True)
        cneg_m = jnp.sum(jnp.where(m, len_, zero), axis=0, keepdims=True)
        cneg = (nrow + one) - cneg_m
        bp = jnp.sum(jnp.where(m, one, zero), axis=0, keepdims=True)

        updp = bmax > mx_v
        cj_v = jnp.where(updp, tp_v + cpos, cj_v)
        mx_v = jnp.where(updp, bmax, mx_v)
        updn = bmin < mn_v
        ck_v = jnp.where(updn, tn_v + cneg, ck_v)
        mn_v = jnp.where(updn, bmin, mn_v)
        tp_v = tp_v + bp
        tn_v = tn_v + (jnp.float32(chunk) - bp)

    mx[...] = mx_v
    mn[...] = mn_v
    cj[...] = cj_v
    ck[...] = ck_v
    tp[...] = tp_v
    tn[...] = tn_v

    @pl.when(c == nc - 1)
    def _fin():
        j_ref[0] = cj[...].astype(jnp.int32) - 1
        k_ref[0] = ck[...].astype(jnp.int32) - 1


def _loss_kernel(xi_ref, xj_ref, xk_ref, sqi_ref,
                 rank_ref, refm_ref, accr, accf):
    st = pl.program_id(0)
    ns = pl.num_programs(0)

    @pl.when(st == 0)
    def _init():
        accr[...] = jnp.zeros(accr.shape, jnp.float32)
        accf[...] = jnp.zeros(accf.shape, jnp.float32)

    xi = xi_ref[...].astype(jnp.float32)
    xj = xj_ref[...].astype(jnp.float32)
    xk = xk_ref[...].astype(jnp.float32)
    sqi = sqi_ref[...]
    sqj = 0.5 * jnp.sum(xj * xj, axis=1, keepdims=True)
    sqk = 0.5 * jnp.sum(xk * xk, axis=1, keepdims=True)
    pij = jnp.sum(xi * xj, axis=1, keepdims=True)
    pik = jnp.sum(xi * xk, axis=1, keepdims=True)
    pjk = jnp.sum(xj * xk, axis=1, keepdims=True)
    eps = jnp.float32(1e-12)
    dap = jnp.sqrt(jnp.maximum(2.0 * (sqi + sqj - pij), eps))
    dan = jnp.sqrt(jnp.maximum(2.0 * (sqi + sqk - pik), eps))
    djk = jnp.sqrt(jnp.maximum(2.0 * (sqj + sqk - pjk), eps))
    rank = jnp.maximum(dap - dan + jnp.float32(_MARGIN), 0.0)
    refv = jnp.abs(dan - djk)
    accr[...] = accr[...] + jnp.sum(rank, axis=0, keepdims=True)[:, :1]
    accf[...] = accf[...] + jnp.sum(refv, axis=0, keepdims=True)[:, :1]

    @pl.when(st == ns - 1)
    def _fin():
        rank_ref[...] = accr[...]
        refm_ref[...] = accf[...]


def kernel(inputs, targets):
    x = inputs.astype(jnp.float32)
    t = targets.astype(jnp.int32)
    n, d = x.shape

    bp_ = min(1024, n)
    x16, sqh = pl.pallas_call(
        _prep_kernel,
        grid=(n // bp_,),
        in_specs=[pl.BlockSpec((bp_, d), lambda i: (i, 0))],
        out_specs=[
            pl.BlockSpec((bp_, d), lambda i: (i, 0)),
            pl.BlockSpec((bp_, 1), lambda i: (i, 0)),
        ],
        out_shape=[
            jax.ShapeDtypeStruct((n, d), jnp.bfloat16),
            jax.ShapeDtypeStruct((n, 1), jnp.float32),
        ],
        compiler_params=pltpu.CompilerParams(
            dimension_semantics=("parallel",),
        ),
        name="prep_norms",
    )(x)

    br = min(2048, n)
    bcnd = min(1024, n)
    nrb = n // br
    ncb = n // bcnd
    ta3 = t.reshape(nrb, 1, br)
    tb3 = t.reshape(ncb, bcnd, 1)
    sqh3 = sqh.reshape(ncb, bcnd, 1)

    j3, k3 = pl.pallas_call(
        _mine_kernel,
        grid=(nrb, ncb),
        in_specs=[
            pl.BlockSpec((br, d), lambda i, c: (i, 0)),
            pl.BlockSpec((bcnd, d), lambda i, c: (c, 0)),
            pl.BlockSpec((1, bcnd, 1), lambda i, c: (c, 0, 0)),
            pl.BlockSpec((1, 1, br), lambda i, c: (i, 0, 0)),
            pl.BlockSpec((1, bcnd, 1), lambda i, c: (c, 0, 0)),
        ],
        out_specs=[
            pl.BlockSpec((1, 1, br), lambda i, c: (i, 0, 0)),
            pl.BlockSpec((1, 1, br), lambda i, c: (i, 0, 0)),
        ],
        out_shape=[
            jax.ShapeDtypeStruct((nrb, 1, br), jnp.int32),
            jax.ShapeDtypeStruct((nrb, 1, br), jnp.int32),
        ],
        scratch_shapes=[
            pltpu.VMEM((1, br), jnp.float32),
            pltpu.VMEM((1, br), jnp.float32),
            pltpu.VMEM((1, br), jnp.float32),
            pltpu.VMEM((1, br), jnp.float32),
            pltpu.VMEM((1, br), jnp.float32),
            pltpu.VMEM((1, br), jnp.float32),
        ],
        compiler_params=pltpu.CompilerParams(
            dimension_semantics=("parallel", "arbitrary"),
        ),
        name="hard_mine",
    )(x16, x16, sqh3, ta3, tb3)

    j = j3.reshape(n)
    k = k3.reshape(n)
    xjk16 = jnp.take(x16, jnp.concatenate([j, k]), axis=0, mode="wrap")

    b2 = min(1024, n)
    ns2 = n // b2
    rank_s, ref_s = pl.pallas_call(
        _loss_kernel,
        grid=(ns2,),
        in_specs=[
            pl.BlockSpec((b2, d), lambda s: (s, 0)),
            pl.BlockSpec((b2, d), lambda s: (s, 0)),
            pl.BlockSpec((b2, d), lambda s, _n=ns2: (s + _n, 0)),
            pl.BlockSpec((b2, 1), lambda s: (s, 0)),
        ],
        out_specs=[
            pl.BlockSpec((1, 1), lambda s: (0, 0)),
            pl.BlockSpec((1, 1), lambda s: (0, 0)),
        ],
        out_shape=[
            jax.ShapeDtypeStruct((1, 1), jnp.float32),
            jax.ShapeDtypeStruct((1, 1), jnp.float32),
        ],
        scratch_shapes=[
            pltpu.VMEM((1, 1), jnp.float32),
            pltpu.VMEM((1, 1), jnp.float32),
        ],
        compiler_params=pltpu.CompilerParams(
            dimension_semantics=("arbitrary",),
        ),
        name="mined_loss",
    )(x16, xjk16, xjk16, sqh)

    return (rank_s[0, 0] + ref_s[0, 0]) / jnp.float32(n)


# speedup vs baseline: 3.5080x; 1.0246x over previous
import jax
import jax.numpy as jnp
from jax.experimental import pallas as pl
from jax.experimental.pallas import tpu as pltpu

_MARGIN = 0.3


def _prep_kernel(x_ref, x16_ref, sqh_ref):
    x = x_ref[...]
    x16_ref[...] = x.astype(jnp.bfloat16)
    sqh_ref[...] = 0.5 * jnp.sum(x * x, axis=1, keepdims=True)


def _mine_kernel(a_ref, b_ref, sqh_ref, ta_ref, tb_ref, j_ref, k_ref,
                 mx, mn, cj, ck, tp, tn):
    c = pl.program_id(1)
    nc = pl.num_programs(1)

    @pl.when(c == 0)
    def _init():
        mx[...] = jnp.full(mx.shape, -jnp.inf, jnp.float32)
        mn[...] = jnp.full(mn.shape, jnp.inf, jnp.float32)
        cj[...] = jnp.zeros(cj.shape, jnp.float32)
        ck[...] = jnp.zeros(ck.shape, jnp.float32)
        tp[...] = jnp.zeros(tp.shape, jnp.float32)
        tn[...] = jnp.zeros(tn.shape, jnp.float32)

    a = a_ref[...]
    ta = ta_ref[0]

    one = jnp.float32(1.0)
    zero = jnp.float32(0.0)
    big = jnp.float32(3.0e38)

    mx_v = mx[...]
    mn_v = mn[...]
    cj_v = cj[...]
    ck_v = ck[...]
    tp_v = tp[...]
    tn_v = tn[...]

    bc_full = b_ref.shape[0]
    chunk = 256
    for u in range(bc_full // chunk):
        b = b_ref[u * chunk:(u + 1) * chunk, :]
        tb = tb_ref[0, u * chunk:(u + 1) * chunk, :]
        sqbh = sqh_ref[0, u * chunk:(u + 1) * chunk, :]

        g = jax.lax.dot_general(b, a, (((1,), (1,)), ((), ())),
                                preferred_element_type=jnp.float32)
        s = sqbh - g

        m = tb == ta
        pos = jnp.where(m, s, -jnp.inf)
        neg = jnp.where(m, jnp.inf, s)
        bmax = jnp.max(pos, axis=0, keepdims=True)
        bmin = jnp.min(neg, axis=0, keepdims=True)

        rowf = jax.lax.broadcasted_iota(jnp.int32, m.shape, 0).astype(jnp.float32)
        prow = jnp.min(jnp.where(pos == bmax, rowf, big), axis=0, keepdims=True)
        nrow = jnp.min(jnp.where(neg == bmin, rowf, big), axis=0, keepdims=True)

        lep = jnp.where(rowf <= prow, one, zero)
        len_ = jnp.where(rowf <= nrow, one, zero)
        cpos = jnp.sum(jnp.where(m, lep, zero), axis=0, keepdims=True)
        cneg_m = jnp.sum(jnp.where(m, len_, zero), axis=0, keepdims=True)
        cneg = (nrow + one) - cneg_m
        bp = jnp.sum(jnp.where(m, one, zero), axis=0, keepdims=True)

        updp = bmax > mx_v
        cj_v = jnp.where(updp, tp_v + cpos, cj_v)
        mx_v = jnp.where(updp, bmax, mx_v)
        updn = bmin < mn_v
        ck_v = jnp.where(updn, tn_v + cneg, ck_v)
        mn_v = jnp.where(updn, bmin, mn_v)
        tp_v = tp_v + bp
        tn_v = tn_v + (jnp.float32(chunk) - bp)

    mx[...] = mx_v
    mn[...] = mn_v
    cj[...] = cj_v
    ck[...] = ck_v
    tp[...] = tp_v
    tn[...] = tn_v

    @pl.when(c == nc - 1)
    def _fin():
        j_ref[0] = cj[...].astype(jnp.int32) - 1
        k_ref[0] = ck[...].astype(jnp.int32) - 1


def _loss_kernel(xi_ref, xj_ref, xk_ref, sqi_ref,
                 rank_ref, refm_ref, accr, accf):
    st = pl.program_id(0)
    ns = pl.num_programs(0)

    @pl.when(st == 0)
    def _init():
        accr[...] = jnp.zeros(accr.shape, jnp.float32)
        accf[...] = jnp.zeros(accf.shape, jnp.float32)

    xi = xi_ref[...]
    xj = xj_ref[...]
    xk = xk_ref[...]
    sqi = sqi_ref[...]
    sqj = 0.5 * jnp.sum(xj * xj, axis=1, keepdims=True)
    sqk = 0.5 * jnp.sum(xk * xk, axis=1, keepdims=True)
    pij = jnp.sum(xi * xj, axis=1, keepdims=True)
    pik = jnp.sum(xi * xk, axis=1, keepdims=True)
    pjk = jnp.sum(xj * xk, axis=1, keepdims=True)
    eps = jnp.float32(1e-12)
    dap = jnp.sqrt(jnp.maximum(2.0 * (sqi + sqj - pij), eps))
    dan = jnp.sqrt(jnp.maximum(2.0 * (sqi + sqk - pik), eps))
    djk = jnp.sqrt(jnp.maximum(2.0 * (sqj + sqk - pjk), eps))
    rank = jnp.maximum(dap - dan + jnp.float32(_MARGIN), 0.0)
    refv = jnp.abs(dan - djk)
    accr[...] = accr[...] + jnp.sum(rank, axis=0, keepdims=True)[:, :1]
    accf[...] = accf[...] + jnp.sum(refv, axis=0, keepdims=True)[:, :1]

    @pl.when(st == ns - 1)
    def _fin():
        rank_ref[...] = accr[...]
        refm_ref[...] = accf[...]


def kernel(inputs, targets):
    x = inputs.astype(jnp.float32)
    t = targets.astype(jnp.int32)
    n, d = x.shape

    bp_ = min(1024, n)
    x16, sqh = pl.pallas_call(
        _prep_kernel,
        grid=(n // bp_,),
        in_specs=[pl.BlockSpec((bp_, d), lambda i: (i, 0))],
        out_specs=[
            pl.BlockSpec((bp_, d), lambda i: (i, 0)),
            pl.BlockSpec((bp_, 1), lambda i: (i, 0)),
        ],
        out_shape=[
            jax.ShapeDtypeStruct((n, d), jnp.bfloat16),
            jax.ShapeDtypeStruct((n, 1), jnp.float32),
        ],
        compiler_params=pltpu.CompilerParams(
            dimension_semantics=("parallel",),
        ),
        name="prep_norms",
    )(x)

    br = min(2048, n)
    bcnd = min(1024, n)
    nrb = n // br
    ncb = n // bcnd
    ta3 = t.reshape(nrb, 1, br)
    tb3 = t.reshape(ncb, bcnd, 1)
    sqh3 = sqh.reshape(ncb, bcnd, 1)

    j3, k3 = pl.pallas_call(
        _mine_kernel,
        grid=(nrb, ncb),
        in_specs=[
            pl.BlockSpec((br, d), lambda i, c: (i, 0)),
            pl.BlockSpec((bcnd, d), lambda i, c: (c, 0)),
            pl.BlockSpec((1, bcnd, 1), lambda i, c: (c, 0, 0)),
            pl.BlockSpec((1, 1, br), lambda i, c: (i, 0, 0)),
            pl.BlockSpec((1, bcnd, 1), lambda i, c: (c, 0, 0)),
        ],
        out_specs=[
            pl.BlockSpec((1, 1, br), lambda i, c: (i, 0, 0)),
            pl.BlockSpec((1, 1, br), lambda i, c: (i, 0, 0)),
        ],
        out_shape=[
            jax.ShapeDtypeStruct((nrb, 1, br), jnp.int32),
            jax.ShapeDtypeStruct((nrb, 1, br), jnp.int32),
        ],
        scratch_shapes=[
            pltpu.VMEM((1, br), jnp.float32),
            pltpu.VMEM((1, br), jnp.float32),
            pltpu.VMEM((1, br), jnp.float32),
            pltpu.VMEM((1, br), jnp.float32),
            pltpu.VMEM((1, br), jnp.float32),
            pltpu.VMEM((1, br), jnp.float32),
        ],
        compiler_params=pltpu.CompilerParams(
            dimension_semantics=("parallel", "arbitrary"),
        ),
        name="hard_mine",
    )(x16, x16, sqh3, ta3, tb3)

    j = j3.reshape(n)
    k = k3.reshape(n)
    xj = jnp.take(x, j, axis=0, mode="wrap")
    xk = jnp.take(x, k, axis=0, mode="wrap")

    b2 = min(1024, n)
    ns2 = n // b2
    rank_s, ref_s = pl.pallas_call(
        _loss_kernel,
        grid=(ns2,),
        in_specs=[
            pl.BlockSpec((b2, d), lambda s: (s, 0)),
            pl.BlockSpec((b2, d), lambda s: (s, 0)),
            pl.BlockSpec((b2, d), lambda s: (s, 0)),
            pl.BlockSpec((b2, 1), lambda s: (s, 0)),
        ],
        out_specs=[
            pl.BlockSpec((1, 1), lambda s: (0, 0)),
            pl.BlockSpec((1, 1), lambda s: (0, 0)),
        ],
        out_shape=[
            jax.ShapeDtypeStruct((1, 1), jnp.float32),
            jax.ShapeDtypeStruct((1, 1), jnp.float32),
        ],
        scratch_shapes=[
            pltpu.VMEM((1, 1), jnp.float32),
            pltpu.VMEM((1, 1), jnp.float32),
        ],
        compiler_params=pltpu.CompilerParams(
            dimension_semantics=("arbitrary",),
        ),
        name="mined_loss",
    )(x, xj, xk, sqh)

    return (rank_s[0, 0] + ref_s[0, 0]) / jnp.float32(n)
